# baseline (device time: 213232 ns/iter reference)
import jax
import jax.numpy as jnp
from jax import lax
from jax.experimental import pallas as pl
from jax.experimental.pallas import tpu as pltpu

N_DEV = 32


def _gelu(z):
    return 0.5 * z * (1.0 + jnp.tanh(0.7978845608 * (z + 0.044715 * z * z * z)))


def kernel(A, B):
    m, k = A.shape
    k2, n = B.shape
    assert k == k2
    rows_per = m // N_DEV

    n_steps = N_DEV - 1

    def body(a_ref, b_ref, out_ref, comm_ref, send_sems, recv_sems):
        my = lax.axis_index("i")
        left = (my - 1) % N_DEV
        right = (my + 1) % N_DEV

        barrier_sem = pltpu.get_barrier_semaphore()
        for nbr in (left, right):
            pl.semaphore_signal(
                barrier_sem, inc=1,
                device_id=(nbr,), device_id_type=pl.DeviceIdType.MESH,
            )
        pl.semaphore_wait(barrier_sem, 2)

        out_ref[:, :] = jnp.dot(
            a_ref[:, :], b_ref[:, :], preferred_element_type=jnp.float32
        )

        for s in range(n_steps):
            send_chunk = (my - s) % N_DEV
            recv_chunk = (my - s - 1) % N_DEV
            rdma = pltpu.make_async_remote_copy(
                src_ref=out_ref.at[pl.ds(send_chunk * rows_per, rows_per), :],
                dst_ref=comm_ref.at[s],
                send_sem=send_sems.at[s],
                recv_sem=recv_sems.at[s],
                device_id=(right,),
                device_id_type=pl.DeviceIdType.MESH,
            )
            rdma.start()
            rdma.wait()
            out_ref[pl.ds(recv_chunk * rows_per, rows_per), :] += comm_ref[s]

        own = (my + 1) % N_DEV
        own_slice = pl.ds(own * rows_per, rows_per)
        out_ref[own_slice, :] = _gelu(out_ref[own_slice, :])

        for s in range(n_steps):
            send_chunk = (my + 1 - s) % N_DEV
            recv_chunk = (my - s) % N_DEV
            t = n_steps + s
            rdma = pltpu.make_async_remote_copy(
                src_ref=out_ref.at[pl.ds(send_chunk * rows_per, rows_per), :],
                dst_ref=comm_ref.at[t],
                send_sem=send_sems.at[t],
                recv_sem=recv_sems.at[t],
                device_id=(right,),
                device_id_type=pl.DeviceIdType.MESH,
            )
            rdma.start()
            rdma.wait()
            out_ref[pl.ds(recv_chunk * rows_per, rows_per), :] = comm_ref[t]

    return pl.pallas_call(
        body,
        out_shape=jax.ShapeDtypeStruct((m, n), jnp.float32),
        in_specs=[
            pl.BlockSpec(memory_space=pltpu.VMEM),
            pl.BlockSpec(memory_space=pltpu.VMEM),
        ],
        out_specs=pl.BlockSpec(memory_space=pltpu.VMEM),
        scratch_shapes=[
            pltpu.VMEM((2 * n_steps, rows_per, n), jnp.float32),
            pltpu.SemaphoreType.DMA((2 * n_steps,)),
            pltpu.SemaphoreType.DMA((2 * n_steps,)),
        ],
        compiler_params=pltpu.CompilerParams(collective_id=0),
    )(A, B)


# device time: 136348 ns/iter; 1.5639x vs baseline; 1.5639x over previous
import jax
import jax.numpy as jnp
from jax import lax
from jax.experimental import pallas as pl
from jax.experimental.pallas import tpu as pltpu

N_DEV = 32
NZ = 4
NP = 8


def _gelu(z):
    return 0.5 * z * (1.0 + jnp.tanh(0.7978845608 * (z + 0.044715 * z * z * z)))


def kernel(A, B):
    m, k = A.shape
    k2, n = B.shape
    assert k == k2
    zrows = m // NZ
    prows = zrows // NP

    def body(a_ref, b_ref, out_ref, zcomm, pcomm, zsend, zrecv, psend, precv):
        my = lax.axis_index("i")
        z = my // NP
        p = my % NP
        z_left = (my - NP) % N_DEV
        z_right = (my + NP) % N_DEV
        p_left = z * NP + (p - 1) % NP
        p_right = z * NP + (p + 1) % NP

        barrier_sem = pltpu.get_barrier_semaphore()
        for nbr in (z_left, z_right, p_left, p_right):
            pl.semaphore_signal(
                barrier_sem, inc=1,
                device_id=(nbr,), device_id_type=pl.DeviceIdType.MESH,
            )
        pl.semaphore_wait(barrier_sem, 4)

        out_ref[:, :] = jnp.dot(
            a_ref[:, :], b_ref[:, :], preferred_element_type=jnp.float32
        )

        for s in range(NZ - 1):
            send_j = (z - s) % NZ
            recv_j = (z - s - 1) % NZ
            rdma = pltpu.make_async_remote_copy(
                src_ref=out_ref.at[pl.ds(send_j * zrows, zrows), :],
                dst_ref=zcomm.at[s],
                send_sem=zsend.at[s],
                recv_sem=zrecv.at[s],
                device_id=(z_right,),
                device_id_type=pl.DeviceIdType.MESH,
            )
            rdma.start()
            rdma.wait()
            out_ref[pl.ds(recv_j * zrows, zrows), :] += zcomm[s]

        own_j = (z + 1) % NZ
        jbase = own_j * zrows

        for s in range(NP - 1):
            send_q = (p - s) % NP
            recv_q = (p - s - 1) % NP
            rdma = pltpu.make_async_remote_copy(
                src_ref=out_ref.at[pl.ds(jbase + send_q * prows, prows), :],
                dst_ref=pcomm.at[s],
                send_sem=psend.at[s],
                recv_sem=precv.at[s],
                device_id=(p_right,),
                device_id_type=pl.DeviceIdType.MESH,
            )
            rdma.start()
            rdma.wait()
            out_ref[pl.ds(jbase + recv_q * prows, prows), :] += pcomm[s]

        own_q = (p + 1) % NP
        own_slice = pl.ds(jbase + own_q * prows, prows)
        out_ref[own_slice, :] = _gelu(out_ref[own_slice, :])

        for s in range(NP - 1):
            t = (NP - 1) + s
            send_q = (p + 1 - s) % NP
            recv_q = (p - s) % NP
            rdma = pltpu.make_async_remote_copy(
                src_ref=out_ref.at[pl.ds(jbase + send_q * prows, prows), :],
                dst_ref=pcomm.at[t],
                send_sem=psend.at[t],
                recv_sem=precv.at[t],
                device_id=(p_right,),
                device_id_type=pl.DeviceIdType.MESH,
            )
            rdma.start()
            rdma.wait()
            out_ref[pl.ds(jbase + recv_q * prows, prows), :] = pcomm[t]

        for s in range(NZ - 1):
            t = (NZ - 1) + s
            send_j = (z + 1 - s) % NZ
            recv_j = (z - s) % NZ
            rdma = pltpu.make_async_remote_copy(
                src_ref=out_ref.at[pl.ds(send_j * zrows, zrows), :],
                dst_ref=zcomm.at[t],
                send_sem=zsend.at[t],
                recv_sem=zrecv.at[t],
                device_id=(z_right,),
                device_id_type=pl.DeviceIdType.MESH,
            )
            rdma.start()
            rdma.wait()
            out_ref[pl.ds(recv_j * zrows, zrows), :] = zcomm[t]

    n_z = 2 * (NZ - 1)
    n_p = 2 * (NP - 1)
    return pl.pallas_call(
        body,
        out_shape=jax.ShapeDtypeStruct((m, n), jnp.float32),
        in_specs=[
            pl.BlockSpec(memory_space=pltpu.VMEM),
            pl.BlockSpec(memory_space=pltpu.VMEM),
        ],
        out_specs=pl.BlockSpec(memory_space=pltpu.VMEM),
        scratch_shapes=[
            pltpu.VMEM((n_z, zrows, n), jnp.float32),
            pltpu.VMEM((n_p, prows, n), jnp.float32),
            pltpu.SemaphoreType.DMA((n_z,)),
            pltpu.SemaphoreType.DMA((n_z,)),
            pltpu.SemaphoreType.DMA((n_p,)),
            pltpu.SemaphoreType.DMA((n_p,)),
        ],
        compiler_params=pltpu.CompilerParams(collective_id=0),
    )(A, B)
